# baseline (speedup 1.0000x reference)
import functools

import jax
import jax.numpy as jnp
from jax import lax
from jax.experimental import pallas as pl
from jax.experimental.pallas import tpu as pltpu
from jax.experimental.pallas import tpu_sc as plsc

NC = 2
NS = 16
L = 16
NW = NC * NS

B = 16384
D = 32
V = 1_000_000
VMAIN = 999_936
VTAIL = V - VMAIN
BPW = B // NW
CHUNK = 128
NCH = BPW // CHUNK

TPB = 31
WS = TPB * 128
NTILES = VMAIN // 128
NCOLS = NTILES // TPB
REGION = NTILES * 1024
HSIZE = 2 * REGION
NSLABS = 2 * 2 * NCOLS
QMAX = NTILES - 1
GSPAN = QMAX * 1024 + 1024

TCW = 124
TCB = TCW * 128
TCG = NTILES // TCW

_mesh = plsc.VectorSubcoreMesh(
    core_axis_name="c", subcore_axis_name="s", num_cores=NC, num_subcores=NS
)


@functools.partial(
    pl.kernel,
    out_type=(
        jax.ShapeDtypeStruct((HSIZE // 1024, 8, 128), jnp.float32),
        jax.ShapeDtypeStruct((HSIZE // 1024, 8, 128), jnp.float32),
    ),
    mesh=_mesh,
    compiler_params=pltpu.CompilerParams(
        needs_layout_passes=False, use_tc_tiling_on_sc=True),
    scratch_types=[
        pltpu.VMEM((TPB, 8, 128), jnp.float32),
        pltpu.SemaphoreType.DMA,
    ],
)
def _repack(ut_hbm, it_hbm, uout, iout, buf, sem):
    wid = lax.axis_index("s") * NC + lax.axis_index("c")

    def slab_body(k, carry):
        t = k // (2 * NCOLS)
        rem = k % (2 * NCOLS)
        r = rem // NCOLS
        c = rem % NCOLS
        row8 = r * 8
        ctile = c * TPB
        tile0 = r * NTILES + ctile

        @pl.when(t == 0)
        def _():
            for j in range(TPB):
                pltpu.async_copy(
                    ut_hbm.at[pl.ds(row8, 8), pl.ds((ctile + j) * 128, 128)],
                    buf.at[j], sem)
            for j in range(TPB):
                pltpu.make_async_copy(
                    ut_hbm.at[pl.ds(row8, 8), pl.ds((ctile + j) * 128, 128)],
                    buf.at[j], sem).wait()
            pltpu.sync_copy(buf, uout.at[pl.ds(tile0, TPB)])

        @pl.when(t == 1)
        def _():
            for j in range(TPB):
                pltpu.async_copy(
                    it_hbm.at[pl.ds(row8, 8), pl.ds((ctile + j) * 128, 128)],
                    buf.at[j], sem)
            for j in range(TPB):
                pltpu.make_async_copy(
                    it_hbm.at[pl.ds(row8, 8), pl.ds((ctile + j) * 128, 128)],
                    buf.at[j], sem).wait()
            pltpu.sync_copy(buf, iout.at[pl.ds(tile0, TPB)])

        return carry

    k0 = (wid * NSLABS) // NW
    k1 = ((wid + 1) * NSLABS) // NW
    lax.fori_loop(k0, k1, slab_body, 0)


def _repack_tc_body(i_ref, o_ref):
    for j in range(TCW):
        o_ref[j] = i_ref[:, pl.ds(j * 128, 128)]


_repack_tc = pl.pallas_call(
    _repack_tc_body,
    grid=(2, TCG),
    in_specs=[pl.BlockSpec((8, TCB), lambda rr, c: (rr + 2, c))],
    out_specs=pl.BlockSpec((TCW, 8, 128), lambda rr, c: (rr * TCG + c, 0, 0)),
    out_shape=jax.ShapeDtypeStruct((HSIZE // 1024, 8, 128), jnp.float32),
)


@functools.partial(
    pl.kernel,
    out_type=jax.ShapeDtypeStruct((B,), jnp.float32),
    mesh=_mesh,
    compiler_params=pltpu.CompilerParams(
        needs_layout_passes=False, use_tc_tiling_on_sc=False),
    scratch_types=[
        pltpu.VMEM((BPW,), jnp.int32),
        pltpu.VMEM((BPW,), jnp.int32),
        pltpu.VMEM((BPW,), jnp.int32),
        pltpu.VMEM((BPW,), jnp.int32),
        pltpu.VMEM((D, BPW), jnp.float32),
        pltpu.VMEM((D, BPW), jnp.float32),
        pltpu.VMEM((VTAIL * D,), jnp.float32),
        pltpu.VMEM((VTAIL * D,), jnp.float32),
        pltpu.VMEM((BPW,), jnp.float32),
        pltpu.SemaphoreType.DMA,
    ],
)
def _cf_ratings(uid_hbm, iid_hbm, utl_hbm, uth_hbm, itl_hbm, ith_hbm,
                tu_hbm, ti_hbm, out_hbm,
                uidx, iidx, upart, ipart, ucols, icols, utail, itail,
                outv, sem):
    wid = lax.axis_index("s") * NC + lax.axis_index("c")
    base = wid * BPW

    pltpu.sync_copy(uid_hbm.at[pl.ds(base, BPW)], uidx)
    pltpu.sync_copy(iid_hbm.at[pl.ds(base, BPW)], iidx)
    pltpu.sync_copy(tu_hbm, utail)
    pltpu.sync_copy(ti_hbm, itail)

    def part_body(g, carry):
        sl = pl.ds(g * L, L)
        for idv, pv in ((uidx, upart), (iidx, ipart)):
            ids = idv[sl]
            q = jnp.minimum(lax.shift_right_logical(ids, 7), QMAX)
            pv[sl] = q * 1024 + lax.bitwise_and(ids, 127)
        outv[pl.ds(g * L, L)] = jnp.zeros((L,), jnp.float32)
        return carry
    lax.fori_loop(0, BPW // L, part_body, 0)

    def make_fire_drain(u_hbm, i_hbm, half):
        def fire(d):
            dd = half * 16 + d
            offs = (d // 8) * REGION + (d % 8) * 128
            for c in range(NCH):
                sl = pl.ds(c * CHUNK, CHUNK)
                pltpu.async_copy(
                    u_hbm.at[pl.ds(offs, GSPAN)].at[upart.at[sl]],
                    ucols.at[dd, sl], sem)
                pltpu.async_copy(
                    i_hbm.at[pl.ds(offs, GSPAN)].at[ipart.at[sl]],
                    icols.at[dd, sl], sem)

        def drain(d):
            dd = half * 16 + d
            offs = (d // 8) * REGION + (d % 8) * 128
            for c in range(NCH):
                sl = pl.ds(c * CHUNK, CHUNK)
                pltpu.make_async_copy(
                    u_hbm.at[pl.ds(offs, GSPAN)].at[upart.at[sl]],
                    ucols.at[dd, sl], sem).wait()
                pltpu.make_async_copy(
                    i_hbm.at[pl.ds(offs, GSPAN)].at[ipart.at[sl]],
                    icols.at[dd, sl], sem).wait()
        return fire, drain

    for half, (uh, ih) in enumerate(((utl_hbm, itl_hbm),
                                     (uth_hbm, ith_hbm))):
        fire, drain = make_fire_drain(uh, ih, half)
        fire(0)
        fire(1)

        def d_loop(d, carry, fire=fire, drain=drain, half=half):
            lax.cond(d < 14, lambda: fire(d + 2), lambda: None)
            drain(d)
            dd = half * 16 + d

            def group_body(g, c2):
                sl = pl.ds(g * L, L)
                outv[sl] = outv[sl] + ucols[dd, sl] * icols[dd, sl]
                return c2
            lax.fori_loop(0, BPW // L, group_body, 0)
            return carry

        lax.fori_loop(0, 16, d_loop, 0)

    def fix_body(g, carry):
        sl = pl.ds(g * L, L)
        uids = uidx[sl]
        iids = iidx[sl]
        um = uids >= VMAIN
        im = iids >= VMAIN
        any_tail = lax.reduce_max(
            jnp.where(um | im, jnp.int32(1), jnp.int32(0)), axes=(0,))

        @pl.when(any_tail > 0)
        def _():
            ubase = jnp.where(um, uids - VMAIN, 0) * D
            ibase = jnp.where(im, iids - VMAIN, 0) * D
            acc = jnp.zeros((L,), jnp.float32)
            for d in range(D):
                tu = plsc.load_gather(utail, [ubase + d])
                ti = plsc.load_gather(itail, [ibase + d])
                u = jnp.where(um, tu, ucols[d, sl])
                v = jnp.where(im, ti, icols[d, sl])
                acc = acc + u * v
            outv[sl] = acc
        return carry
    lax.fori_loop(0, BPW // L, fix_body, 0)

    pltpu.sync_copy(outv, out_hbm.at[pl.ds(base, BPW)])


def kernel(user_ids, item_ids, user_table, item_table):
    uid = user_ids.astype(jnp.int32)
    iid = item_ids.astype(jnp.int32)
    utl, itl = _repack(user_table.T, item_table.T)
    uth = _repack_tc(user_table.T)
    ith = _repack_tc(item_table.T)
    tail_u = user_table[VMAIN:].reshape(VTAIL * D)
    tail_i = item_table[VMAIN:].reshape(VTAIL * D)
    return _cf_ratings(uid, iid,
                       utl.reshape(HSIZE), uth.reshape(HSIZE),
                       itl.reshape(HSIZE), ith.reshape(HSIZE),
                       tail_u, tail_i)
